# Initial kernel scaffold; baseline (speedup 1.0000x reference)
import jax
import jax.numpy as jnp
from jax.experimental import pallas as pl


def kernel(edge_sim, edge_rat, x, W1s, b1s, W1r, b1r, W1f, b1f, W2s, b2s, W2r, b2r, W2f, b2f):
    raise NotImplementedError("write your pallas kernel here")


# speedup vs baseline: 2.0902x; 2.0902x over previous
import jax
import jax.numpy as jnp
from jax import lax
from jax.experimental import pallas as pl
from jax.experimental.pallas import tpu as pltpu
from jax.experimental.pallas import tpu_sc as plsc

N = 10000
E = 320000
D_IN = 128
D_HID = 200
D_PAD = 256

NC = 2
NS = 16
HALF = D_PAD // NC
CHUNK = 128
CPT = 160
IB = 16
NIB = CPT // IB
E_PAD = NS * CPT * CHUNK
N_PAD = 10240
RPT = N_PAD // NS
ZR = 16
ROW_BLK = 1280


def _prep_edges(edge):
    src = edge[0].astype(jnp.int32)
    dst = edge[1].astype(jnp.int32)
    pad = E_PAD - E
    src = jnp.concatenate([src, jnp.zeros((pad,), jnp.int32)])
    dst = jnp.concatenate([dst, jnp.full((pad,), N, jnp.int32)])
    return src.reshape(NS, CPT, CHUNK), dst.reshape(NS, CPT, CHUNK)


def _sc_agg_body(hs, hr, ss, sd, rs, rd, agg,
                 src_v, dst_v, rows_v, zb_v, acc_sh, sem):
    cid = lax.axis_index("c")
    sid = lax.axis_index("s")
    base = sid * RPT

    zv = jnp.zeros((16,), jnp.float32)
    for i in range(ZR):
        for k in range(HALF // 16):
            zb_v[i, pl.ds(k * 16, 16)] = zv

    def zcopy(i, c):
        pltpu.sync_copy(zb_v, acc_sh.at[pl.ds(base + i * ZR, ZR)])
        return c
    lax.fori_loop(0, RPT // ZR, zcopy, 0)

    plsc.subcore_barrier()

    def run(col):
        def one_set(src_hbm, dst_hbm, h_hbm):
            def blk(b, c):
                pltpu.sync_copy(src_hbm.at[sid, pl.ds(b * IB, IB)], src_v)
                pltpu.sync_copy(dst_hbm.at[sid, pl.ds(b * IB, IB)], dst_v)

                def body(j, c2):
                    pltpu.async_copy(
                        h_hbm.at[src_v.at[j], pl.ds(col, HALF)], rows_v, sem
                    ).wait()
                    pltpu.sync_copy(rows_v, acc_sh.at[dst_v.at[j]], add=True)
                    return c2
                lax.fori_loop(0, IB, body, c)
                return c
            lax.fori_loop(0, NIB, blk, 0)

        one_set(ss, sd, hs)
        one_set(rs, rd, hr)
        plsc.subcore_barrier()
        pltpu.sync_copy(acc_sh.at[pl.ds(base, RPT)],
                        agg.at[pl.ds(base, RPT), pl.ds(col, HALF)])

    @pl.when(cid == 0)
    def _():
        run(0)

    @pl.when(cid == 1)
    def _():
        run(HALF)


_sc_agg = pl.kernel(
    _sc_agg_body,
    out_type=jax.ShapeDtypeStruct((N_PAD, D_PAD), jnp.float32),
    mesh=plsc.VectorSubcoreMesh(core_axis_name="c", subcore_axis_name="s"),
    scratch_types=[
        pltpu.VMEM((IB, CHUNK), jnp.int32),
        pltpu.VMEM((IB, CHUNK), jnp.int32),
        pltpu.VMEM((CHUNK, HALF), jnp.float32),
        pltpu.VMEM((ZR, HALF), jnp.float32),
        pltpu.VMEM_SHARED((N_PAD, HALF), jnp.float32),
        pltpu.SemaphoreType.DMA,
    ],
)


def _mm3_kernel(x_ref, ws_ref, wr_ref, wf_ref, bs_ref, br_ref, bf_ref,
                os_ref, or_ref, of_ref):
    xb = x_ref[...]
    os_ref[...] = jnp.dot(xb, ws_ref[...], preferred_element_type=jnp.float32) + bs_ref[...]
    or_ref[...] = jnp.dot(xb, wr_ref[...], preferred_element_type=jnp.float32) + br_ref[...]
    of_ref[...] = jnp.dot(xb, wf_ref[...], preferred_element_type=jnp.float32) + bf_ref[...]


def _relu_mm3_kernel(f_ref, agg_ref, ws_ref, wr_ref, wf_ref,
                     bs_ref, br_ref, bf_ref, os_ref, or_ref, of_ref):
    h = jnp.maximum(f_ref[...] + agg_ref[...], 0.0)
    os_ref[...] = jnp.dot(h, ws_ref[...], preferred_element_type=jnp.float32) + bs_ref[...]
    or_ref[...] = jnp.dot(h, wr_ref[...], preferred_element_type=jnp.float32) + br_ref[...]
    of_ref[...] = jnp.dot(h, wf_ref[...], preferred_element_type=jnp.float32) + bf_ref[...]


def _combine_kernel(f_ref, agg_ref, o_ref):
    o_ref[...] = f_ref[...] + agg_ref[...]


def _mm3(lead, Ws, Wr, Wf, bs, br, bf, K, relu_combine=False):
    grid = (N_PAD // ROW_BLK,)
    lead_specs = [pl.BlockSpec((ROW_BLK, K), lambda i: (i, 0))] * len(lead)
    w_specs = [pl.BlockSpec((K, D_PAD), lambda i: (0, 0))] * 3
    b_specs = [pl.BlockSpec((1, D_PAD), lambda i: (0, 0))] * 3
    body = _relu_mm3_kernel if relu_combine else _mm3_kernel
    return pl.pallas_call(
        body,
        grid=grid,
        in_specs=lead_specs + w_specs + b_specs,
        out_specs=[pl.BlockSpec((ROW_BLK, D_PAD), lambda i: (i, 0))] * 3,
        out_shape=[jax.ShapeDtypeStruct((N_PAD, D_PAD), jnp.float32)] * 3,
    )(*lead, Ws, Wr, Wf, bs, br, bf)


def _combine(f, agg):
    grid = (N_PAD // ROW_BLK,)
    spec = pl.BlockSpec((ROW_BLK, D_PAD), lambda i: (i, 0))
    return pl.pallas_call(
        _combine_kernel,
        grid=grid,
        in_specs=[spec] * 2,
        out_specs=spec,
        out_shape=jax.ShapeDtypeStruct((N_PAD, D_PAD), jnp.float32),
    )(f, agg)


def _pad_w(W, K_PAD):
    return jnp.pad(W, ((0, K_PAD - W.shape[0]), (0, D_PAD - W.shape[1])))


def _pad_b(b):
    return jnp.pad(b, (0, D_PAD - b.shape[0])).reshape(1, D_PAD)


def kernel(edge_sim, edge_rat, x,
           W1s, b1s, W1r, b1r, W1f, b1f,
           W2s, b2s, W2r, b2r, W2f, b2f):
    ss, sd = _prep_edges(edge_sim)
    rs, rd = _prep_edges(edge_rat)
    x_p = jnp.pad(x, ((0, N_PAD - N), (0, 0)))

    hs1, hr1, f1 = _mm3((x_p,), _pad_w(W1s, D_IN), _pad_w(W1r, D_IN),
                        _pad_w(W1f, D_IN), _pad_b(b1s), _pad_b(b1r),
                        _pad_b(b1f), D_IN)
    agg1 = _sc_agg(hs1, hr1, ss, sd, rs, rd)
    hs2, hr2, f2 = _mm3((f1, agg1), _pad_w(W2s, D_PAD), _pad_w(W2r, D_PAD),
                        _pad_w(W2f, D_PAD), _pad_b(b2s), _pad_b(b2r),
                        _pad_b(b2f), D_PAD, relu_combine=True)
    agg2 = _sc_agg(hs2, hr2, ss, sd, rs, rd)
    out = _combine(f2, agg2)
    return out[:N, :D_HID]


# speedup vs baseline: 2.2530x; 1.0779x over previous
import jax
import jax.numpy as jnp
from jax import lax
from jax.experimental import pallas as pl
from jax.experimental.pallas import tpu as pltpu
from jax.experimental.pallas import tpu_sc as plsc

N = 10000
E = 320000
D_IN = 128
D_HID = 200
D_PAD = 256

NC = 2
NS = 16
HALF = D_PAD // NC
CHUNK = 128
CPT = 160
IB = 4
NIB = CPT // IB
E_PAD = NS * CPT * CHUNK
N_PAD = 10240
RPT = N_PAD // NS
ZR = 16
ROW_BLK = 1280


def _prep_edges(edge):
    src = edge[0].astype(jnp.int32)
    dst = edge[1].astype(jnp.int32)
    pad = E_PAD - E
    src = jnp.concatenate([src, jnp.zeros((pad,), jnp.int32)])
    dst = jnp.concatenate([dst, jnp.full((pad,), N, jnp.int32)])
    return src.reshape(NS, CPT, CHUNK), dst.reshape(NS, CPT, CHUNK)


def _sc_agg_body(hs, hr, ss, sd, rs, rd, agg,
                 src_v, dst_v, rows0, rows1, zb_v, acc_sh,
                 gsem0, gsem1, ssem0, ssem1):
    rows = (rows0, rows1)
    gsem = (gsem0, gsem1)
    ssem = (ssem0, ssem1)
    cid = lax.axis_index("c")
    sid = lax.axis_index("s")
    base = sid * RPT

    zv = jnp.zeros((16,), jnp.float32)
    for i in range(ZR):
        for k in range(HALF // 16):
            zb_v[i, pl.ds(k * 16, 16)] = zv

    def zcopy(i, c):
        pltpu.sync_copy(zb_v, acc_sh.at[pl.ds(base + i * ZR, ZR)])
        return c
    lax.fori_loop(0, RPT // ZR, zcopy, 0)

    plsc.subcore_barrier()

    def run(col):
        def one_set(src_hbm, dst_hbm, h_hbm):
            def blk(b, c):
                pltpu.sync_copy(src_hbm.at[sid, pl.ds(b * IB, IB)], src_v)
                pltpu.sync_copy(dst_hbm.at[sid, pl.ds(b * IB, IB)], dst_v)

                gd = [None] * IB
                sd_ = [None] * IB
                for j in range(IB):
                    bi = j & 1
                    if j >= 2:
                        sd_[j - 2].wait()
                    gd[j] = pltpu.async_copy(
                        h_hbm.at[src_v.at[j], pl.ds(col, HALF)], rows[bi],
                        gsem[bi])
                    if j >= 1:
                        gd[j - 1].wait()
                        sd_[j - 1] = pltpu.async_copy(
                            rows[(j - 1) & 1], acc_sh.at[dst_v.at[j - 1]],
                            ssem[(j - 1) & 1], add=True)
                gd[IB - 1].wait()
                sd_[IB - 1] = pltpu.async_copy(
                    rows[(IB - 1) & 1], acc_sh.at[dst_v.at[IB - 1]],
                    ssem[(IB - 1) & 1], add=True)
                sd_[IB - 2].wait()
                sd_[IB - 1].wait()
                return c
            lax.fori_loop(0, NIB, blk, 0)

        one_set(ss, sd, hs)
        one_set(rs, rd, hr)
        plsc.subcore_barrier()
        pltpu.sync_copy(acc_sh.at[pl.ds(base, RPT)],
                        agg.at[pl.ds(base, RPT), pl.ds(col, HALF)])

    @pl.when(cid == 0)
    def _():
        run(0)

    @pl.when(cid == 1)
    def _():
        run(HALF)


_sc_agg = pl.kernel(
    _sc_agg_body,
    out_type=jax.ShapeDtypeStruct((N_PAD, D_PAD), jnp.float32),
    mesh=plsc.VectorSubcoreMesh(core_axis_name="c", subcore_axis_name="s"),
    scratch_types=[
        pltpu.VMEM((IB, CHUNK), jnp.int32),
        pltpu.VMEM((IB, CHUNK), jnp.int32),
        pltpu.VMEM((CHUNK, HALF), jnp.float32),
        pltpu.VMEM((CHUNK, HALF), jnp.float32),
        pltpu.VMEM((ZR, HALF), jnp.float32),
        pltpu.VMEM_SHARED((N_PAD, HALF), jnp.float32),
        pltpu.SemaphoreType.DMA,
        pltpu.SemaphoreType.DMA,
        pltpu.SemaphoreType.DMA,
        pltpu.SemaphoreType.DMA,
    ],
)


def _mm3_kernel(x_ref, ws_ref, wr_ref, wf_ref, bs_ref, br_ref, bf_ref,
                os_ref, or_ref, of_ref):
    xb = x_ref[...]
    os_ref[...] = jnp.dot(xb, ws_ref[...], preferred_element_type=jnp.float32) + bs_ref[...]
    or_ref[...] = jnp.dot(xb, wr_ref[...], preferred_element_type=jnp.float32) + br_ref[...]
    of_ref[...] = jnp.dot(xb, wf_ref[...], preferred_element_type=jnp.float32) + bf_ref[...]


def _relu_mm3_kernel(f_ref, agg_ref, ws_ref, wr_ref, wf_ref,
                     bs_ref, br_ref, bf_ref, os_ref, or_ref, of_ref):
    h = jnp.maximum(f_ref[...] + agg_ref[...], 0.0)
    os_ref[...] = jnp.dot(h, ws_ref[...], preferred_element_type=jnp.float32) + bs_ref[...]
    or_ref[...] = jnp.dot(h, wr_ref[...], preferred_element_type=jnp.float32) + br_ref[...]
    of_ref[...] = jnp.dot(h, wf_ref[...], preferred_element_type=jnp.float32) + bf_ref[...]


def _combine_kernel(f_ref, agg_ref, o_ref):
    o_ref[...] = f_ref[...] + agg_ref[...]


def _mm3(lead, Ws, Wr, Wf, bs, br, bf, K, relu_combine=False):
    grid = (N_PAD // ROW_BLK,)
    lead_specs = [pl.BlockSpec((ROW_BLK, K), lambda i: (i, 0))] * len(lead)
    w_specs = [pl.BlockSpec((K, D_PAD), lambda i: (0, 0))] * 3
    b_specs = [pl.BlockSpec((1, D_PAD), lambda i: (0, 0))] * 3
    body = _relu_mm3_kernel if relu_combine else _mm3_kernel
    return pl.pallas_call(
        body,
        grid=grid,
        in_specs=lead_specs + w_specs + b_specs,
        out_specs=[pl.BlockSpec((ROW_BLK, D_PAD), lambda i: (i, 0))] * 3,
        out_shape=[jax.ShapeDtypeStruct((N_PAD, D_PAD), jnp.float32)] * 3,
    )(*lead, Ws, Wr, Wf, bs, br, bf)


def _combine(f, agg):
    grid = (N_PAD // ROW_BLK,)
    spec = pl.BlockSpec((ROW_BLK, D_PAD), lambda i: (i, 0))
    return pl.pallas_call(
        _combine_kernel,
        grid=grid,
        in_specs=[spec] * 2,
        out_specs=spec,
        out_shape=jax.ShapeDtypeStruct((N_PAD, D_PAD), jnp.float32),
    )(f, agg)


def _pad_w(W, K_PAD):
    return jnp.pad(W, ((0, K_PAD - W.shape[0]), (0, D_PAD - W.shape[1])))


def _pad_b(b):
    return jnp.pad(b, (0, D_PAD - b.shape[0])).reshape(1, D_PAD)


def kernel(edge_sim, edge_rat, x,
           W1s, b1s, W1r, b1r, W1f, b1f,
           W2s, b2s, W2r, b2r, W2f, b2f):
    ss, sd = _prep_edges(edge_sim)
    rs, rd = _prep_edges(edge_rat)
    x_p = jnp.pad(x, ((0, N_PAD - N), (0, 0)))

    hs1, hr1, f1 = _mm3((x_p,), _pad_w(W1s, D_IN), _pad_w(W1r, D_IN),
                        _pad_w(W1f, D_IN), _pad_b(b1s), _pad_b(b1r),
                        _pad_b(b1f), D_IN)
    agg1 = _sc_agg(hs1, hr1, ss, sd, rs, rd)
    hs2, hr2, f2 = _mm3((f1, agg1), _pad_w(W2s, D_PAD), _pad_w(W2r, D_PAD),
                        _pad_w(W2f, D_PAD), _pad_b(b2s), _pad_b(b2r),
                        _pad_b(b2f), D_PAD, relu_combine=True)
    agg2 = _sc_agg(hs2, hr2, ss, sd, rs, rd)
    out = _combine(f2, agg2)
    return out[:N, :D_HID]
